# speedup vs baseline: 2.0355x; 2.0355x over previous
import functools

import jax
import jax.numpy as jnp
from jax import lax
from jax.experimental import pallas as pl
from jax.experimental.pallas import tpu as pltpu
from jax.experimental.pallas import tpu_sc as plsc

_NC = 2
_NS = 16
_NW = _NC * _NS
_CH = 32


@functools.lru_cache(maxsize=None)
def _make_gather(n_idx: int, dim: int):
    bpw = n_idx // _NW
    nchunk = bpw // _CH
    mesh = plsc.VectorSubcoreMesh(core_axis_name="c", subcore_axis_name="s")

    @functools.partial(
        pl.kernel,
        out_type=jax.ShapeDtypeStruct((n_idx, dim), jnp.float32),
        mesh=mesh,
        scratch_types=[
            pltpu.VMEM((bpw,), jnp.int32),
            pltpu.VMEM((_CH, dim), jnp.float32),
            pltpu.VMEM((_CH, dim), jnp.float32),
            pltpu.SemaphoreType.DMA,
            pltpu.SemaphoreType.DMA,
        ],
    )
    def grab(table_hbm, idx_hbm, out_hbm, idx_v, buf0, buf1, sem0, sem1):
        wid = lax.axis_index("s") * _NC + lax.axis_index("c")
        base = wid * bpw
        pltpu.sync_copy(idx_hbm.at[pl.ds(base, bpw)], idx_v)

        bufs = (buf0, buf1)
        sems = (sem0, sem1)
        copies = [None, None]
        copies[0] = pltpu.async_copy(
            table_hbm.at[idx_v.at[pl.ds(0, _CH)]], bufs[0], sems[0])
        for c in range(nchunk):
            b = c % 2
            if c + 1 < nchunk:
                copies[1 - b] = pltpu.async_copy(
                    table_hbm.at[idx_v.at[pl.ds((c + 1) * _CH, _CH)]],
                    bufs[1 - b], sems[1 - b])
            copies[b].wait()
            pltpu.sync_copy(bufs[b], out_hbm.at[pl.ds(base + c * _CH, _CH)])

    return grab


def kernel(position, position_encoding):
    batch, seq = position.shape
    dim = position_encoding.shape[1]
    idx = position.reshape(-1).astype(jnp.int32)
    table = position_encoding.astype(jnp.float32)
    out = _make_gather(idx.shape[0], dim)(table, idx)
    return out.reshape(batch, seq, dim)
